# baseline (speedup 1.0000x reference)
import jax
import jax.numpy as jnp
from jax import lax
from jax.experimental import pallas as pl

N = 8192
D = 768
MT = 16
MY = 32
BN = 2048
NS = 4
BS = BN // NS


def _routed_softmax(xb, tb, w, b):
    logits = jnp.dot(xb.astype(jnp.bfloat16), w,
                     preferred_element_type=jnp.float32)
    logits = logits + b
    col_task = lax.broadcasted_iota(jnp.int32, (BS, MT * MY), 1) // MY
    masked = jnp.where(col_task == tb, logits, 0.0)
    comp = (lax.broadcasted_iota(jnp.int32, (MT * MY, MY), 0) % MY ==
            lax.broadcasted_iota(jnp.int32, (MT * MY, MY), 1))
    zl = jnp.dot(masked, comp.astype(jnp.float32),
                 preferred_element_type=jnp.float32)
    m = jnp.max(zl, axis=1, keepdims=True)
    p = jnp.exp(zl - m)
    return p / jnp.sum(p, axis=1, keepdims=True)


def _body(*refs):
    x_refs = refs[:NS]
    t_refs = refs[NS:2 * NS]
    w_ref, b_ref, o_ref = refs[2 * NS], refs[2 * NS + 1], refs[2 * NS + 2]
    w = w_ref[...]
    b = b_ref[...]
    for s in range(NS):
        o_ref[pl.ds(s * BS, BS), :] = _routed_softmax(
            x_refs[s][...], t_refs[s][...], w, b)


def kernel(x, t, W, b):
    w2 = W.reshape(MT * MY, D).T.astype(jnp.bfloat16)
    b2 = b.reshape(1, MT * MY)
    t2 = t.reshape(N, 1)
    x_specs = [pl.BlockSpec((BS, D), lambda i, s=s: (NS * i + s, 0))
               for s in range(NS)]
    t_specs = [pl.BlockSpec((BS, 1), lambda i, s=s: (NS * i + s, 0))
               for s in range(NS)]
    return pl.pallas_call(
        _body,
        grid=(N // BN,),
        in_specs=x_specs + t_specs + [
            pl.BlockSpec((D, MT * MY), lambda i: (0, 0)),
            pl.BlockSpec((1, MT * MY), lambda i: (0, 0)),
        ],
        out_specs=pl.BlockSpec((BN, MY), lambda i: (i, 0)),
        out_shape=jax.ShapeDtypeStruct((N, MY), jnp.float32),
    )(*([x] * NS), *([t2] * NS), w2, b2)
